# Initial kernel scaffold; baseline (speedup 1.0000x reference)
import functools

import jax
import jax.numpy as jnp
from jax import lax
from jax.experimental import pallas as pl
from jax.experimental.pallas import tpu as pltpu
from jax.experimental.pallas import tpu_sc as plsc

_DIMS = [119, 5, 12, 12, 10, 6, 6, 2, 2]
_NT = len(_DIMS)
_OFFS = [sum(_DIMS[:i]) for i in range(_NT)]
_ROWS = sum(_DIMS)
_D = 128
_N = 100000
_BR = 32
_NBLK = _N // _BR
_NW = 32
_BASE = _NBLK // _NW
_REM = _NBLK % _NW

_mesh = plsc.VectorSubcoreMesh(
    core_axis_name="c", subcore_axis_name="s", num_cores=2, num_subcores=16
)


@functools.partial(
    pl.kernel,
    out_type=jax.ShapeDtypeStruct((_N * _D,), jnp.float32),
    mesh=_mesh,
    scratch_types=[
        pltpu.VMEM((_ROWS * _D,), jnp.float32),
        pltpu.VMEM((2, _BR * _NT), jnp.int32),
        pltpu.VMEM((2, _BR * _D), jnp.float32),
        pltpu.SemaphoreType.DMA((2,)),
        pltpu.SemaphoreType.DMA((2,)),
    ],
)
def _sc_embed(xf, wf, out, table_v, xv, ov, sem_idx, sem_out):
    wid = lax.axis_index("s") * 2 + lax.axis_index("c")
    nb = jnp.where(wid < _REM, _BASE + 1, _BASE)
    b0 = wid * _BASE + jnp.minimum(wid, _REM)

    pltpu.sync_copy(wf, table_v)

    pltpu.async_copy(
        xf.at[pl.ds(b0 * (_BR * _NT), _BR * _NT)], xv.at[0], sem_idx.at[0]
    )

    iota16 = lax.broadcasted_iota(jnp.int32, (16,), 0)

    @pl.loop(0, nb)
    def _block(g):
        slot = lax.rem(g, 2)
        nslot = 1 - slot
        b = b0 + g

        pltpu.make_async_copy(
            xf.at[pl.ds(b * (_BR * _NT), _BR * _NT)], xv.at[slot], sem_idx.at[slot]
        ).wait()

        @pl.when(g + 1 < nb)
        def _():
            pltpu.async_copy(
                xf.at[pl.ds((b + 1) * (_BR * _NT), _BR * _NT)],
                xv.at[nslot],
                sem_idx.at[nslot],
            )

        @pl.when(g >= 2)
        def _():
            pltpu.make_async_copy(
                ov.at[slot],
                out.at[pl.ds((b - 2) * (_BR * _D), _BR * _D)],
                sem_out.at[slot],
            ).wait()

        xblk = xv.at[slot]
        oblk = ov.at[slot]
        for sub in range(_BR // 16):
            rowv = iota16 + sub * 16
            rb = rowv * _NT
            addrs = []
            for c in range(_NT):
                xi = plsc.load_gather(xblk, [rb + c])
                addrs.append((xi + _OFFS[c]) * _D)
            ob = rowv * _D

            @plsc.parallel_loop(0, _D, unroll=4)
            def _dim(j):
                acc = plsc.load_gather(table_v, [addrs[0] + j])
                for c in range(1, _NT):
                    acc = acc + plsc.load_gather(table_v, [addrs[c] + j])
                plsc.store_scatter(oblk, [ob + j], acc)

        pltpu.async_copy(
            oblk, out.at[pl.ds(b * (_BR * _D), _BR * _D)], sem_out.at[slot]
        )

    for k in (2, 1):
        bk = b0 + nb - k
        slotk = lax.rem(nb - k, 2)
        pltpu.make_async_copy(
            ov.at[slotk],
            out.at[pl.ds(bk * (_BR * _D), _BR * _D)],
            sem_out.at[slotk],
        ).wait()


def kernel(x, W0, W1, W2, W3, W4, W5, W6, W7, W8):
    wf = jnp.concatenate([W0, W1, W2, W3, W4, W5, W6, W7, W8], axis=0).reshape(-1)
    xf = x.reshape(-1)
    out = _sc_embed(xf, wf)
    return out.reshape(_N, _D)


# speedup vs baseline: 1.0373x; 1.0373x over previous
import functools

import jax
import jax.numpy as jnp
from jax import lax
from jax.experimental import pallas as pl
from jax.experimental.pallas import tpu as pltpu
from jax.experimental.pallas import tpu_sc as plsc

_DIMS = [119, 5, 12, 12, 10, 6, 6, 2, 2]
_NT = len(_DIMS)
_OFFS = [sum(_DIMS[:i]) for i in range(_NT)]
_ROWS = sum(_DIMS)
_D = 128
_N = 100000
_BR = 32
_NBLK = _N // _BR
_NW = 32
_BASE = _NBLK // _NW
_REM = _NBLK % _NW
_MAXR = (_BASE + 1) * _BR
_NPAD = _NW * _BASE * _BR + _REM * _BR + _BR

_mesh = plsc.VectorSubcoreMesh(
    core_axis_name="c", subcore_axis_name="s", num_cores=2, num_subcores=16
)


@functools.partial(
    pl.kernel,
    out_type=jax.ShapeDtypeStruct((_N * _D,), jnp.float32),
    mesh=_mesh,
    scratch_types=[
        pltpu.VMEM((_ROWS * _D,), jnp.float32),
        pltpu.VMEM((_NT, _MAXR), jnp.int32),
        pltpu.VMEM((16 * _D,), jnp.float32),
        pltpu.VMEM((16 * _D,), jnp.float32),
        pltpu.SemaphoreType.DMA,
        pltpu.SemaphoreType.DMA,
        pltpu.SemaphoreType.DMA,
    ],
    compiler_params=pltpu.CompilerParams(
        use_tc_tiling_on_sc=False, needs_layout_passes=False
    ),
)
def _sc_embed(xt, wf, out, table_v, xch, ov0, ov1, sem_x, sem_o0, sem_o1):
    wid = lax.axis_index("s") * 2 + lax.axis_index("c")
    nblocks = jnp.where(wid < _REM, _BASE + 1, _BASE)
    b0 = wid * _BASE + jnp.minimum(wid, _REM)
    row0 = b0 * _BR

    pltpu.async_copy(xt.at[:, pl.ds(row0, _MAXR)], xch, sem_x)
    pltpu.sync_copy(wf, table_v)
    pltpu.make_async_copy(xt.at[:, pl.ds(row0, _MAXR)], xch, sem_x).wait()

    iota16 = lax.broadcasted_iota(jnp.int32, (16,), 0)
    ob = iota16 * _D

    def compute_half(local_r, oref):
        addrs = []
        for c in range(_NT):
            xi = xch[c, pl.ds(local_r, 16)]
            addrs.append((xi + _OFFS[c]) * _D)

        @plsc.parallel_loop(0, _D, unroll=4)
        def _dim(j):
            acc = plsc.load_gather(table_v, [addrs[0] + j])
            for c in range(1, _NT):
                acc = acc + plsc.load_gather(table_v, [addrs[c] + j])
            plsc.store_scatter(oref, [ob + j], acc)

    @pl.loop(0, nblocks)
    def _pair(t):
        b = b0 + t
        for half, (oref, sem) in enumerate(((ov0, sem_o0), (ov1, sem_o1))):
            @pl.when(t >= 1)
            def _():
                pltpu.make_async_copy(
                    oref,
                    out.at[pl.ds((b - 1) * (_BR * _D) + half * (16 * _D), 16 * _D)],
                    sem,
                ).wait()

            compute_half(t * _BR + half * 16, oref)
            pltpu.async_copy(
                oref,
                out.at[pl.ds(b * (_BR * _D) + half * (16 * _D), 16 * _D)],
                sem,
            )

    blast = b0 + nblocks - 1
    for half, (oref, sem) in enumerate(((ov0, sem_o0), (ov1, sem_o1))):
        pltpu.make_async_copy(
            oref,
            out.at[pl.ds(blast * (_BR * _D) + half * (16 * _D), 16 * _D)],
            sem,
        ).wait()


def kernel(x, W0, W1, W2, W3, W4, W5, W6, W7, W8):
    wf = jnp.concatenate([W0, W1, W2, W3, W4, W5, W6, W7, W8], axis=0).reshape(-1)
    xpad = jnp.concatenate(
        [x, jnp.zeros((_NPAD - _N, _NT), dtype=x.dtype)], axis=0
    )
    xt = xpad.T
    out = _sc_embed(xt, wf)
    return out.reshape(_N, _D)


# speedup vs baseline: 1.1814x; 1.1389x over previous
import functools

import jax
import jax.numpy as jnp
from jax import lax
from jax.experimental import pallas as pl
from jax.experimental.pallas import tpu as pltpu
from jax.experimental.pallas import tpu_sc as plsc

_DIMS = [119, 5, 12, 12, 10, 6, 6, 2, 2]
_NT = len(_DIMS)
_OFFS = [sum(_DIMS[:i]) for i in range(_NT)]
_ROWS = sum(_DIMS)
_D = 128
_N = 100000
_BR = 32
_NBLK = _N // _BR
_NW = 32
_BASE = _NBLK // _NW
_REM = _NBLK % _NW
_MAXR = (_BASE + 1) * _BR
_NPAD = _NW * _BASE * _BR + _REM * _BR + _BR

_mesh = plsc.VectorSubcoreMesh(
    core_axis_name="c", subcore_axis_name="s", num_cores=2, num_subcores=16
)


@functools.partial(
    pl.kernel,
    out_type=jax.ShapeDtypeStruct((_N * _D,), jnp.float32),
    mesh=_mesh,
    scratch_types=[
        pltpu.VMEM((_ROWS * _D,), jnp.float32),
        pltpu.VMEM((_NT, _MAXR), jnp.int32),
        pltpu.VMEM((16 * _D,), jnp.float32),
        pltpu.VMEM((16 * _D,), jnp.float32),
        pltpu.SemaphoreType.DMA,
        pltpu.SemaphoreType.DMA,
        pltpu.SemaphoreType.DMA,
    ],
    compiler_params=pltpu.CompilerParams(
        use_tc_tiling_on_sc=False, needs_layout_passes=False
    ),
)
def _sc_embed(xt, wf, out, table_v, xch, ov0, ov1, sem_x, sem_o0, sem_o1):
    wid = lax.axis_index("s") * 2 + lax.axis_index("c")
    nblocks = jnp.where(wid < _REM, _BASE + 1, _BASE)
    b0 = wid * _BASE + jnp.minimum(wid, _REM)
    row0 = b0 * _BR

    pltpu.async_copy(xt.at[:, pl.ds(row0, _MAXR)], xch, sem_x)
    pltpu.sync_copy(wf, table_v)
    pltpu.make_async_copy(xt.at[:, pl.ds(row0, _MAXR)], xch, sem_x).wait()

    iota16 = lax.broadcasted_iota(jnp.int32, (16,), 0)
    ob = iota16 * _D

    def compute_half(local_r, oref):
        addrs = []
        for c in range(_NT):
            xi = xch[c, pl.ds(local_r, 16)]
            addrs.append((xi + _OFFS[c]) * _D)

        @plsc.parallel_loop(0, _D, unroll=8)
        def _dim(j):
            acc = plsc.load_gather(table_v, [addrs[0] + j])
            for c in range(1, _NT):
                acc = acc + plsc.load_gather(table_v, [addrs[c] + j])
            plsc.store_scatter(oref, [ob + j], acc)

    @pl.loop(0, nblocks)
    def _pair(t):
        b = b0 + t
        for half, (oref, sem) in enumerate(((ov0, sem_o0), (ov1, sem_o1))):
            @pl.when(t >= 1)
            def _():
                pltpu.make_async_copy(
                    oref,
                    out.at[pl.ds((b - 1) * (_BR * _D) + half * (16 * _D), 16 * _D)],
                    sem,
                ).wait()

            compute_half(t * _BR + half * 16, oref)
            pltpu.async_copy(
                oref,
                out.at[pl.ds(b * (_BR * _D) + half * (16 * _D), 16 * _D)],
                sem,
            )

    blast = b0 + nblocks - 1
    for half, (oref, sem) in enumerate(((ov0, sem_o0), (ov1, sem_o1))):
        pltpu.make_async_copy(
            oref,
            out.at[pl.ds(blast * (_BR * _D) + half * (16 * _D), 16 * _D)],
            sem,
        ).wait()


def kernel(x, W0, W1, W2, W3, W4, W5, W6, W7, W8):
    wf = jnp.concatenate([W0, W1, W2, W3, W4, W5, W6, W7, W8], axis=0).reshape(-1)
    xpad = jnp.concatenate(
        [x, jnp.zeros((_NPAD - _N, _NT), dtype=x.dtype)], axis=0
    )
    xt = xpad.T
    out = _sc_embed(xt, wf)
    return out.reshape(_N, _D)


# speedup vs baseline: 6.7266x; 5.6938x over previous
import functools

import jax
import jax.numpy as jnp
from jax import lax
from jax.experimental import pallas as pl
from jax.experimental.pallas import tpu as pltpu
from jax.experimental.pallas import tpu_sc as plsc

_DIMS = [119, 5, 12, 12, 10, 6, 6, 2, 2]
_NT = len(_DIMS)
_OFFS = [sum(_DIMS[:i]) for i in range(_NT)]
_ROWS = sum(_DIMS)
_D = 128
_N = 100000
_BR = 32
_NBLK = _N // _BR
_NW = 32
_BASE = _NBLK // _NW
_REM = _NBLK % _NW
_MAXR = (_BASE + 1) * _BR
_NPAD = _NW * _BASE * _BR + _REM * _BR + _BR
_CH = _MAXR * _NT

_mesh = plsc.VectorSubcoreMesh(
    core_axis_name="c", subcore_axis_name="s", num_cores=2, num_subcores=16
)


@functools.partial(
    pl.kernel,
    out_type=jax.ShapeDtypeStruct((_N * _D,), jnp.float32),
    mesh=_mesh,
    scratch_types=[
        pltpu.VMEM((_ROWS * _D,), jnp.float32),
        pltpu.VMEM((_CH + 16,), jnp.int32),
        pltpu.VMEM((16 * _D,), jnp.float32),
        pltpu.VMEM((16 * _D,), jnp.float32),
        pltpu.SemaphoreType.DMA,
        pltpu.SemaphoreType.DMA,
        pltpu.SemaphoreType.DMA,
    ],
    compiler_params=pltpu.CompilerParams(
        use_tc_tiling_on_sc=False, needs_layout_passes=False
    ),
)
def _sc_embed(xf, wf, out, table_v, xch, ov0, ov1, sem_x, sem_o0, sem_o1):
    wid = lax.axis_index("s") * 2 + lax.axis_index("c")
    nblocks = jnp.where(wid < _REM, _BASE + 1, _BASE)
    b0 = wid * _BASE + jnp.minimum(wid, _REM)

    pltpu.async_copy(xf.at[pl.ds(b0 * _BR * _NT, _CH)], xch.at[pl.ds(0, _CH)], sem_x)
    pltpu.sync_copy(wf, table_v)
    pltpu.make_async_copy(
        xf.at[pl.ds(b0 * _BR * _NT, _CH)], xch.at[pl.ds(0, _CH)], sem_x
    ).wait()

    def compute_half(local_r, oref):
        @plsc.parallel_loop(0, 16, unroll=2)
        def _row(r):
            lr = local_r + r
            xrow = xch[pl.ds(lr * _NT, 16)]
            bases = []
            for c in range(_NT):
                bases.append((xrow[c] + _OFFS[c]) * _D)
            for j0 in range(0, _D, 16):
                acc = table_v[pl.ds(bases[0] + j0, 16)]
                for c in range(1, _NT):
                    acc = acc + table_v[pl.ds(bases[c] + j0, 16)]
                oref[pl.ds(r * _D + j0, 16)] = acc

    @pl.loop(0, nblocks)
    def _pair(t):
        b = b0 + t
        for half, (oref, sem) in enumerate(((ov0, sem_o0), (ov1, sem_o1))):
            @pl.when(t >= 1)
            def _():
                pltpu.make_async_copy(
                    oref,
                    out.at[pl.ds((b - 1) * (_BR * _D) + half * (16 * _D), 16 * _D)],
                    sem,
                ).wait()

            compute_half(t * _BR + half * 16, oref)
            pltpu.async_copy(
                oref,
                out.at[pl.ds(b * (_BR * _D) + half * (16 * _D), 16 * _D)],
                sem,
            )

    blast = b0 + nblocks - 1
    for half, (oref, sem) in enumerate(((ov0, sem_o0), (ov1, sem_o1))):
        pltpu.make_async_copy(
            oref,
            out.at[pl.ds(blast * (_BR * _D) + half * (16 * _D), 16 * _D)],
            sem,
        ).wait()


def kernel(x, W0, W1, W2, W3, W4, W5, W6, W7, W8):
    wf = jnp.concatenate([W0, W1, W2, W3, W4, W5, W6, W7, W8], axis=0).reshape(-1)
    xpad = jnp.concatenate(
        [x, jnp.zeros((_NPAD - _N, _NT), dtype=x.dtype)], axis=0
    )
    xf = xpad.reshape(-1)
    out = _sc_embed(xf, wf)
    return out.reshape(_N, _D)


# speedup vs baseline: 8.5111x; 1.2653x over previous
import functools

import jax
import jax.numpy as jnp
from jax import lax
from jax.experimental import pallas as pl
from jax.experimental.pallas import tpu as pltpu
from jax.experimental.pallas import tpu_sc as plsc

_DIMS = [119, 5, 12, 12, 10, 6, 6, 2, 2]
_NT = len(_DIMS)
_OFFS = [sum(_DIMS[:i]) for i in range(_NT)]
_ROWS = sum(_DIMS)
_D = 128
_N = 100000
_BR = 32
_NBLK = _N // _BR
_NW = 32
_BASE = _NBLK // _NW
_REM = _NBLK % _NW
_MAXR = (_BASE + 1) * _BR
_NPAD = _NW * _BASE * _BR + _REM * _BR + _BR
_CH = _MAXR * _NT

_mesh = plsc.VectorSubcoreMesh(
    core_axis_name="c", subcore_axis_name="s", num_cores=2, num_subcores=16
)

_GROUPS = [
    (5, 12, _OFFS[1], _OFFS[2], 119),
    (12, 10, _OFFS[3], _OFFS[4], 179),
    (6, 6, _OFFS[5], _OFFS[6], 299),
    (2, 2, _OFFS[7], _OFFS[8], 335),
]
_CROWS = 339


@functools.partial(
    pl.kernel,
    out_type=jax.ShapeDtypeStruct((_N * _D,), jnp.float32),
    mesh=_mesh,
    scratch_types=[
        pltpu.VMEM((_ROWS * _D,), jnp.float32),
        pltpu.VMEM((_CROWS * _D,), jnp.float32),
        pltpu.VMEM((_CH + 16,), jnp.int32),
        pltpu.VMEM((16 * _D,), jnp.float32),
        pltpu.VMEM((16 * _D,), jnp.float32),
        pltpu.SemaphoreType.DMA,
        pltpu.SemaphoreType.DMA,
        pltpu.SemaphoreType.DMA,
        pltpu.SemaphoreType.DMA,
    ],
    compiler_params=pltpu.CompilerParams(
        use_tc_tiling_on_sc=False, needs_layout_passes=False
    ),
)
def _sc_embed(xf, wf, out, raw_v, comb_v, xch, ov0, ov1, sem_x, sem_g0, sem_o0, sem_o1):
    wid = lax.axis_index("s") * 2 + lax.axis_index("c")
    nblocks = jnp.where(wid < _REM, _BASE + 1, _BASE)
    b0 = wid * _BASE + jnp.minimum(wid, _REM)

    pltpu.async_copy(xf.at[pl.ds(b0 * _BR * _NT, _CH)], xch.at[pl.ds(0, _CH)], sem_x)
    pltpu.async_copy(
        wf.at[pl.ds(0, 119 * _D)], comb_v.at[pl.ds(0, 119 * _D)], sem_g0
    )
    pltpu.sync_copy(wf, raw_v)

    for dA, dB, offA, offB, gbase in _GROUPS:

        @pl.loop(0, dA)
        def _a(a, _dB=dB, _offA=offA, _offB=offB, _gbase=gbase):
            va = [raw_v[pl.ds((_offA + a) * _D + k * 16, 16)] for k in range(8)]
            rowbase = (_gbase + a * _dB) * _D

            @plsc.parallel_loop(0, _dB, unroll=2)
            def _b(b):
                src = (_offB + b) * _D
                dst = rowbase + b * _D
                for k in range(8):
                    comb_v[pl.ds(dst + k * 16, 16)] = va[k] + raw_v[
                        pl.ds(src + k * 16, 16)
                    ]

    pltpu.make_async_copy(
        wf.at[pl.ds(0, 119 * _D)], comb_v.at[pl.ds(0, 119 * _D)], sem_g0
    ).wait()
    pltpu.make_async_copy(
        xf.at[pl.ds(b0 * _BR * _NT, _CH)], xch.at[pl.ds(0, _CH)], sem_x
    ).wait()

    def compute_half(local_r, oref):
        @plsc.parallel_loop(0, 16, unroll=2)
        def _row(r):
            lr = local_r + r
            xrow = xch[pl.ds(lr * _NT, 16)]
            bases = [
                xrow[0] * _D,
                (xrow[1] * 12 + xrow[2] + 119) * _D,
                (xrow[3] * 10 + xrow[4] + 179) * _D,
                (xrow[5] * 6 + xrow[6] + 299) * _D,
                (xrow[7] * 2 + xrow[8] + 335) * _D,
            ]
            for j0 in range(0, _D, 16):
                acc = comb_v[pl.ds(bases[0] + j0, 16)]
                for g in range(1, 5):
                    acc = acc + comb_v[pl.ds(bases[g] + j0, 16)]
                oref[pl.ds(r * _D + j0, 16)] = acc

    @pl.loop(0, nblocks)
    def _pair(t):
        b = b0 + t
        for half, (oref, sem) in enumerate(((ov0, sem_o0), (ov1, sem_o1))):
            @pl.when(t >= 1)
            def _():
                pltpu.make_async_copy(
                    oref,
                    out.at[pl.ds((b - 1) * (_BR * _D) + half * (16 * _D), 16 * _D)],
                    sem,
                ).wait()

            compute_half(t * _BR + half * 16, oref)
            pltpu.async_copy(
                oref,
                out.at[pl.ds(b * (_BR * _D) + half * (16 * _D), 16 * _D)],
                sem,
            )

    blast = b0 + nblocks - 1
    for half, (oref, sem) in enumerate(((ov0, sem_o0), (ov1, sem_o1))):
        pltpu.make_async_copy(
            oref,
            out.at[pl.ds(blast * (_BR * _D) + half * (16 * _D), 16 * _D)],
            sem,
        ).wait()


def kernel(x, W0, W1, W2, W3, W4, W5, W6, W7, W8):
    wf = jnp.concatenate([W0, W1, W2, W3, W4, W5, W6, W7, W8], axis=0).reshape(-1)
    xpad = jnp.concatenate(
        [x, jnp.zeros((_NPAD - _N, _NT), dtype=x.dtype)], axis=0
    )
    xf = xpad.reshape(-1)
    out = _sc_embed(xf, wf)
    return out.reshape(_N, _D)


# speedup vs baseline: 8.5148x; 1.0004x over previous
import functools

import jax
import jax.numpy as jnp
from jax import lax
from jax.experimental import pallas as pl
from jax.experimental.pallas import tpu as pltpu
from jax.experimental.pallas import tpu_sc as plsc

_DIMS = [119, 5, 12, 12, 10, 6, 6, 2, 2]
_NT = len(_DIMS)
_OFFS = [sum(_DIMS[:i]) for i in range(_NT)]
_ROWS = sum(_DIMS)
_D = 128
_N = 100000
_BR = 32
_NBLK = _N // _BR
_NW = 32
_BASE = _NBLK // _NW
_REM = _NBLK % _NW
_MAXR = (_BASE + 1) * _BR
_NPAD = _NW * _BASE * _BR + _REM * _BR + _BR
_CH = _MAXR * _NT

_mesh = plsc.VectorSubcoreMesh(
    core_axis_name="c", subcore_axis_name="s", num_cores=2, num_subcores=16
)

_GROUPS = [
    (5, 12, _OFFS[1], _OFFS[2], 119),
    (12, 10, _OFFS[3], _OFFS[4], 179),
    (6, 6, _OFFS[5], _OFFS[6], 299),
    (2, 2, _OFFS[7], _OFFS[8], 335),
]
_CROWS = 339


@functools.partial(
    pl.kernel,
    out_type=jax.ShapeDtypeStruct((_N * _D,), jnp.float32),
    mesh=_mesh,
    scratch_types=[
        pltpu.VMEM((_ROWS * _D,), jnp.float32),
        pltpu.VMEM((_CROWS * _D,), jnp.float32),
        pltpu.VMEM((_CH + 16,), jnp.int32),
        pltpu.VMEM((16 * _D,), jnp.float32),
        pltpu.VMEM((16 * _D,), jnp.float32),
        pltpu.SemaphoreType.DMA,
        pltpu.SemaphoreType.DMA,
        pltpu.SemaphoreType.DMA,
        pltpu.SemaphoreType.DMA,
    ],
    compiler_params=pltpu.CompilerParams(
        use_tc_tiling_on_sc=False, needs_layout_passes=False
    ),
)
def _sc_embed(xf, wf, out, raw_v, comb_v, xch, ov0, ov1, sem_x, sem_g0, sem_o0, sem_o1):
    wid = lax.axis_index("s") * 2 + lax.axis_index("c")
    nblocks = jnp.where(wid < _REM, _BASE + 1, _BASE)
    b0 = wid * _BASE + jnp.minimum(wid, _REM)

    pltpu.async_copy(xf.at[pl.ds(b0 * _BR * _NT, _CH)], xch.at[pl.ds(0, _CH)], sem_x)
    pltpu.async_copy(
        wf.at[pl.ds(0, 119 * _D)], comb_v.at[pl.ds(0, 119 * _D)], sem_g0
    )
    pltpu.sync_copy(wf, raw_v)

    for dA, dB, offA, offB, gbase in _GROUPS:

        @pl.loop(0, dA)
        def _a(a, _dB=dB, _offA=offA, _offB=offB, _gbase=gbase):
            va = [raw_v[pl.ds((_offA + a) * _D + k * 16, 16)] for k in range(8)]
            rowbase = (_gbase + a * _dB) * _D

            @plsc.parallel_loop(0, _dB, unroll=2)
            def _b(b):
                src = (_offB + b) * _D
                dst = rowbase + b * _D
                for k in range(8):
                    comb_v[pl.ds(dst + k * 16, 16)] = va[k] + raw_v[
                        pl.ds(src + k * 16, 16)
                    ]

    pltpu.make_async_copy(
        wf.at[pl.ds(0, 119 * _D)], comb_v.at[pl.ds(0, 119 * _D)], sem_g0
    ).wait()
    pltpu.make_async_copy(
        xf.at[pl.ds(b0 * _BR * _NT, _CH)], xch.at[pl.ds(0, _CH)], sem_x
    ).wait()

    def compute_half(local_r, oref):
        @plsc.parallel_loop(0, 16, unroll=4)
        def _row(r):
            lr = local_r + r
            xrow = xch[pl.ds(lr * _NT, 16)]
            bases = [
                xrow[0] * _D,
                (xrow[1] * 12 + xrow[2] + 119) * _D,
                (xrow[3] * 10 + xrow[4] + 179) * _D,
                (xrow[5] * 6 + xrow[6] + 299) * _D,
                (xrow[7] * 2 + xrow[8] + 335) * _D,
            ]
            for j0 in range(0, _D, 16):
                acc = comb_v[pl.ds(bases[0] + j0, 16)]
                for g in range(1, 5):
                    acc = acc + comb_v[pl.ds(bases[g] + j0, 16)]
                oref[pl.ds(r * _D + j0, 16)] = acc

    @pl.loop(0, nblocks)
    def _pair(t):
        b = b0 + t
        for half, (oref, sem) in enumerate(((ov0, sem_o0), (ov1, sem_o1))):
            @pl.when(t >= 1)
            def _():
                pltpu.make_async_copy(
                    oref,
                    out.at[pl.ds((b - 1) * (_BR * _D) + half * (16 * _D), 16 * _D)],
                    sem,
                ).wait()

            compute_half(t * _BR + half * 16, oref)
            pltpu.async_copy(
                oref,
                out.at[pl.ds(b * (_BR * _D) + half * (16 * _D), 16 * _D)],
                sem,
            )

    blast = b0 + nblocks - 1
    for half, (oref, sem) in enumerate(((ov0, sem_o0), (ov1, sem_o1))):
        pltpu.make_async_copy(
            oref,
            out.at[pl.ds(blast * (_BR * _D) + half * (16 * _D), 16 * _D)],
            sem,
        ).wait()


def kernel(x, W0, W1, W2, W3, W4, W5, W6, W7, W8):
    wf = jnp.concatenate([W0, W1, W2, W3, W4, W5, W6, W7, W8], axis=0).reshape(-1)
    xpad = jnp.concatenate(
        [x, jnp.zeros((_NPAD - _N, _NT), dtype=x.dtype)], axis=0
    )
    xf = xpad.reshape(-1)
    out = _sc_embed(xf, wf)
    return out.reshape(_N, _D)


# speedup vs baseline: 9.9006x; 1.1627x over previous
import functools

import jax
import jax.numpy as jnp
from jax import lax
from jax.experimental import pallas as pl
from jax.experimental.pallas import tpu as pltpu
from jax.experimental.pallas import tpu_sc as plsc

_DIMS = [119, 5, 12, 12, 10, 6, 6, 2, 2]
_NT = len(_DIMS)
_OFFS = [sum(_DIMS[:i]) for i in range(_NT)]
_ROWS = sum(_DIMS)
_D = 128
_N = 100000
_BR = 32
_NBLK = _N // _BR
_NW = 32
_BASE = _NBLK // _NW
_REM = _NBLK % _NW
_MAXR = (_BASE + 1) * _BR
_NPAD = _NW * _BASE * _BR + _REM * _BR + _BR
_CH = _MAXR * _NT

_mesh = plsc.VectorSubcoreMesh(
    core_axis_name="c", subcore_axis_name="s", num_cores=2, num_subcores=16
)

_GROUPS = [
    (5, 12, _OFFS[1], _OFFS[2], 119),
    (12, 10, _OFFS[3], _OFFS[4], 179),
    (6, 6, _OFFS[5], _OFFS[6], 299),
    (2, 2, _OFFS[7], _OFFS[8], 335),
]
_CROWS = 339


@functools.partial(
    pl.kernel,
    out_type=jax.ShapeDtypeStruct((_N, _D), jnp.float32),
    mesh=_mesh,
    scratch_types=[
        pltpu.VMEM((_ROWS * _D,), jnp.float32),
        pltpu.VMEM((_CROWS * _D,), jnp.float32),
        pltpu.VMEM((_CH + 16,), jnp.int32),
        pltpu.VMEM((16, _D), jnp.float32),
        pltpu.VMEM((16, _D), jnp.float32),
        pltpu.SemaphoreType.DMA,
        pltpu.SemaphoreType.DMA,
        pltpu.SemaphoreType.DMA,
        pltpu.SemaphoreType.DMA,
    ],
    compiler_params=pltpu.CompilerParams(
        use_tc_tiling_on_sc=False, needs_layout_passes=False
    ),
)
def _sc_embed(xf, wf, out, raw_v, comb_v, xch, ov0, ov1, sem_x, sem_g0, sem_o0, sem_o1):
    wid = lax.axis_index("s") * 2 + lax.axis_index("c")
    nblocks = jnp.where(wid < _REM, _BASE + 1, _BASE)
    b0 = wid * _BASE + jnp.minimum(wid, _REM)

    @pl.when(nblocks == _BASE + 1)
    def _():
        pltpu.async_copy(
            xf.at[pl.ds(b0 * _BR * _NT, (_BASE + 1) * _BR * _NT)],
            xch.at[pl.ds(0, (_BASE + 1) * _BR * _NT)],
            sem_x,
        )

    @pl.when(nblocks == _BASE)
    def _():
        pltpu.async_copy(
            xf.at[pl.ds(b0 * _BR * _NT, _BASE * _BR * _NT)],
            xch.at[pl.ds(0, _BASE * _BR * _NT)],
            sem_x,
        )
    pltpu.async_copy(
        wf.at[pl.ds(0, 119 * _D)], comb_v.at[pl.ds(0, 119 * _D)], sem_g0
    )
    pltpu.sync_copy(wf, raw_v)

    for dA, dB, offA, offB, gbase in _GROUPS:

        @pl.loop(0, dA)
        def _a(a, _dB=dB, _offA=offA, _offB=offB, _gbase=gbase):
            va = [raw_v[pl.ds((_offA + a) * _D + k * 16, 16)] for k in range(8)]
            rowbase = (_gbase + a * _dB) * _D

            @plsc.parallel_loop(0, _dB, unroll=2)
            def _b(b):
                src = (_offB + b) * _D
                dst = rowbase + b * _D
                for k in range(8):
                    comb_v[pl.ds(dst + k * 16, 16)] = va[k] + raw_v[
                        pl.ds(src + k * 16, 16)
                    ]

    pltpu.make_async_copy(
        wf.at[pl.ds(0, 119 * _D)], comb_v.at[pl.ds(0, 119 * _D)], sem_g0
    ).wait()

    @pl.when(nblocks == _BASE + 1)
    def _():
        pltpu.make_async_copy(
            xf.at[pl.ds(b0 * _BR * _NT, (_BASE + 1) * _BR * _NT)],
            xch.at[pl.ds(0, (_BASE + 1) * _BR * _NT)],
            sem_x,
        ).wait()

    @pl.when(nblocks == _BASE)
    def _():
        pltpu.make_async_copy(
            xf.at[pl.ds(b0 * _BR * _NT, _BASE * _BR * _NT)],
            xch.at[pl.ds(0, _BASE * _BR * _NT)],
            sem_x,
        ).wait()

    def compute_half(local_r, oref):
        @plsc.parallel_loop(0, 16, unroll=4)
        def _row(r):
            lr = local_r + r
            xrow = xch[pl.ds(lr * _NT, 16)]
            bases = [
                xrow[0] * _D,
                (xrow[1] * 12 + xrow[2] + 119) * _D,
                (xrow[3] * 10 + xrow[4] + 179) * _D,
                (xrow[5] * 6 + xrow[6] + 299) * _D,
                (xrow[7] * 2 + xrow[8] + 335) * _D,
            ]
            for j0 in range(0, _D, 16):
                acc = comb_v[pl.ds(bases[0] + j0, 16)]
                for g in range(1, 5):
                    acc = acc + comb_v[pl.ds(bases[g] + j0, 16)]
                oref[r, pl.ds(j0, 16)] = acc

    @pl.loop(0, nblocks)
    def _pair(t):
        b = b0 + t
        for half, (oref, sem) in enumerate(((ov0, sem_o0), (ov1, sem_o1))):
            @pl.when(t >= 1)
            def _():
                pltpu.make_async_copy(
                    oref, out.at[pl.ds((b - 1) * _BR + half * 16, 16)], sem
                ).wait()

            compute_half(t * _BR + half * 16, oref)
            pltpu.async_copy(oref, out.at[pl.ds(b * _BR + half * 16, 16)], sem)

    blast = b0 + nblocks - 1
    for half, (oref, sem) in enumerate(((ov0, sem_o0), (ov1, sem_o1))):
        pltpu.make_async_copy(
            oref, out.at[pl.ds(blast * _BR + half * 16, 16)], sem
        ).wait()


def kernel(x, W0, W1, W2, W3, W4, W5, W6, W7, W8):
    wf = jnp.concatenate([W0, W1, W2, W3, W4, W5, W6, W7, W8], axis=0).reshape(-1)
    xf = x.reshape(-1)
    return _sc_embed(xf, wf)


# speedup vs baseline: 10.7662x; 1.0874x over previous
import functools

import jax
import jax.numpy as jnp
from jax import lax
from jax.experimental import pallas as pl
from jax.experimental.pallas import tpu as pltpu
from jax.experimental.pallas import tpu_sc as plsc

_DIMS = [119, 5, 12, 12, 10, 6, 6, 2, 2]
_NT = len(_DIMS)
_OFFS = [sum(_DIMS[:i]) for i in range(_NT)]
_ROWS = sum(_DIMS)
_D = 128
_N = 100000
_BR = 32
_NBLK = _N // _BR
_NW = 32
_BASE = _NBLK // _NW
_REM = _NBLK % _NW
_MAXR = (_BASE + 1) * _BR
_NPAD = _NW * _BASE * _BR + _REM * _BR + _BR
_CH = _MAXR * _NT

_mesh = plsc.VectorSubcoreMesh(
    core_axis_name="c", subcore_axis_name="s", num_cores=2, num_subcores=16
)

_GROUPS = [
    (5, 12, _OFFS[1], _OFFS[2], 119),
    (12, 10, _OFFS[3], _OFFS[4], 179),
]
_QBASE = 299
_CROWS = 443


@functools.partial(
    pl.kernel,
    out_type=jax.ShapeDtypeStruct((_N, _D), jnp.float32),
    mesh=_mesh,
    scratch_types=[
        pltpu.VMEM((_ROWS * _D,), jnp.float32),
        pltpu.VMEM((_CROWS * _D,), jnp.float32),
        pltpu.VMEM((_CH + 16,), jnp.int32),
        pltpu.VMEM((16, _D), jnp.float32),
        pltpu.VMEM((16, _D), jnp.float32),
        pltpu.VMEM((4 * _D,), jnp.float32),
        pltpu.SemaphoreType.DMA,
        pltpu.SemaphoreType.DMA,
        pltpu.SemaphoreType.DMA,
        pltpu.SemaphoreType.DMA,
    ],
    compiler_params=pltpu.CompilerParams(
        use_tc_tiling_on_sc=False, needs_layout_passes=False
    ),
)
def _sc_embed(
    xf, wf, out, raw_v, comb_v, xch, ov0, ov1, t4_v, sem_x, sem_g0, sem_o0, sem_o1
):
    wid = lax.axis_index("s") * 2 + lax.axis_index("c")
    nblocks = jnp.where(wid < _REM, _BASE + 1, _BASE)
    b0 = wid * _BASE + jnp.minimum(wid, _REM)

    @pl.when(nblocks == _BASE + 1)
    def _():
        pltpu.async_copy(
            xf.at[pl.ds(b0 * _BR * _NT, (_BASE + 1) * _BR * _NT)],
            xch.at[pl.ds(0, (_BASE + 1) * _BR * _NT)],
            sem_x,
        )

    @pl.when(nblocks == _BASE)
    def _():
        pltpu.async_copy(
            xf.at[pl.ds(b0 * _BR * _NT, _BASE * _BR * _NT)],
            xch.at[pl.ds(0, _BASE * _BR * _NT)],
            sem_x,
        )
    pltpu.async_copy(
        wf.at[pl.ds(0, 119 * _D)], comb_v.at[pl.ds(0, 119 * _D)], sem_g0
    )
    pltpu.sync_copy(wf, raw_v)

    for dA, dB, offA, offB, gbase in _GROUPS:

        @pl.loop(0, dA)
        def _a(a, _dB=dB, _offA=offA, _offB=offB, _gbase=gbase):
            va = [raw_v[pl.ds((_offA + a) * _D + k * 16, 16)] for k in range(8)]
            rowbase = (_gbase + a * _dB) * _D

            @plsc.parallel_loop(0, _dB, unroll=2)
            def _b(b):
                src = (_offB + b) * _D
                dst = rowbase + b * _D
                for k in range(8):
                    comb_v[pl.ds(dst + k * 16, 16)] = va[k] + raw_v[
                        pl.ds(src + k * 16, 16)
                    ]

    for c in range(2):
        for d in range(2):
            for k in range(8):
                t4_v[pl.ds((c * 2 + d) * _D + k * 16, 16)] = (
                    raw_v[pl.ds((_OFFS[7] + c) * _D + k * 16, 16)]
                    + raw_v[pl.ds((_OFFS[8] + d) * _D + k * 16, 16)]
                )

    @pl.loop(0, 6)
    def _qa(a):
        va = [raw_v[pl.ds((_OFFS[5] + a) * _D + k * 16, 16)] for k in range(8)]

        @plsc.parallel_loop(0, 6, unroll=2)
        def _qb(b):
            vab = [
                va[k] + raw_v[pl.ds((_OFFS[6] + b) * _D + k * 16, 16)]
                for k in range(8)
            ]
            base = (_QBASE + (a * 6 + b) * 4) * _D
            for cd in range(4):
                for k in range(8):
                    comb_v[pl.ds(base + cd * _D + k * 16, 16)] = vab[k] + t4_v[
                        pl.ds(cd * _D + k * 16, 16)
                    ]

    pltpu.make_async_copy(
        wf.at[pl.ds(0, 119 * _D)], comb_v.at[pl.ds(0, 119 * _D)], sem_g0
    ).wait()

    @pl.when(nblocks == _BASE + 1)
    def _():
        pltpu.make_async_copy(
            xf.at[pl.ds(b0 * _BR * _NT, (_BASE + 1) * _BR * _NT)],
            xch.at[pl.ds(0, (_BASE + 1) * _BR * _NT)],
            sem_x,
        ).wait()

    @pl.when(nblocks == _BASE)
    def _():
        pltpu.make_async_copy(
            xf.at[pl.ds(b0 * _BR * _NT, _BASE * _BR * _NT)],
            xch.at[pl.ds(0, _BASE * _BR * _NT)],
            sem_x,
        ).wait()

    def compute_half(local_r, oref):
        @plsc.parallel_loop(0, 16, unroll=4)
        def _row(r):
            lr = local_r + r
            xrow = xch[pl.ds(lr * _NT, 16)]
            bases = [
                xrow[0] * _D,
                (xrow[1] * 12 + xrow[2] + 119) * _D,
                (xrow[3] * 10 + xrow[4] + 179) * _D,
                ((xrow[5] * 6 + xrow[6]) * 4 + xrow[7] * 2 + xrow[8] + _QBASE) * _D,
            ]
            for j0 in range(0, _D, 16):
                acc = comb_v[pl.ds(bases[0] + j0, 16)]
                for g in range(1, 4):
                    acc = acc + comb_v[pl.ds(bases[g] + j0, 16)]
                oref[r, pl.ds(j0, 16)] = acc

    @pl.loop(0, nblocks)
    def _pair(t):
        b = b0 + t
        for half, (oref, sem) in enumerate(((ov0, sem_o0), (ov1, sem_o1))):
            @pl.when(t >= 1)
            def _():
                pltpu.make_async_copy(
                    oref, out.at[pl.ds((b - 1) * _BR + half * 16, 16)], sem
                ).wait()

            compute_half(t * _BR + half * 16, oref)
            pltpu.async_copy(oref, out.at[pl.ds(b * _BR + half * 16, 16)], sem)

    blast = b0 + nblocks - 1
    for half, (oref, sem) in enumerate(((ov0, sem_o0), (ov1, sem_o1))):
        pltpu.make_async_copy(
            oref, out.at[pl.ds(blast * _BR + half * 16, 16)], sem
        ).wait()


def kernel(x, W0, W1, W2, W3, W4, W5, W6, W7, W8):
    wf = jnp.concatenate([W0, W1, W2, W3, W4, W5, W6, W7, W8], axis=0).reshape(-1)
    xf = x.reshape(-1)
    return _sc_embed(xf, wf)
